# baseline (speedup 1.0000x reference)
import functools

import jax
import jax.numpy as jnp
from jax import lax
from jax.experimental import pallas as pl
from jax.experimental.pallas import tpu as pltpu
from jax.experimental.pallas import tpu_sc as plsc

EMBED_DIM = 64
ITEM_NUM = 1000000
HALF_ITEMS = ITEM_NUM // 2
BATCH = 16384
N_TAGS = 20

NC = 2
NS = 16
NW = NC * NS
B_PER_W = BATCH // NW
CHUNK = 32
N_CHUNKS = B_PER_W // CHUNK
ROWS_PER_CHUNK = CHUNK * N_TAGS
GATHER = 128
G_PER_CHUNK = ROWS_PER_CHUNK // GATHER
L = 16

TW = 32768
TGRID = -(-ITEM_NUM // TW)
PAIR_ROWS = TGRID * TW // 2


def _transpose_body(x, o):
    xv = x[...]
    parts = [jnp.swapaxes(xv[:, i * 128:(i + 1) * 128], 0, 1)
             for i in range(TW // 128)]
    o[...] = jnp.concatenate(
        [jnp.concatenate(parts[2 * r:2 * r + 2], axis=1)
         for r in range(TW // 256)], axis=0)


def _tags_body(tags_emb, tags_idx, tags_pad, out,
               idx_tags_v, pad_v, tag_buf, out_buf, sem_a, sem_b):
    wid = lax.axis_index("s") * NC + lax.axis_index("c")

    pltpu.sync_copy(tags_idx.at[wid], idx_tags_v)
    pltpu.sync_copy(tags_pad.at[wid], pad_v)

    def issue(c, buf_off, sem):
        for j in range(G_PER_CHUNK):
            pltpu.async_copy(
                tags_emb.at[idx_tags_v.at[c * G_PER_CHUNK + j]],
                tag_buf.at[pl.ds(buf_off + j * GATHER, GATHER)], sem)

    def drain(sem):
        for _ in range(G_PER_CHUNK):
            pltpu.make_async_copy(
                tags_emb.at[pl.ds(0, GATHER)],
                tag_buf.at[pl.ds(0, GATHER)], sem).wait()

    def compute(c, buf_off):
        def elem_body(e, _):
            g = c * CHUNK + e
            p0 = pad_v[g, pl.ds(0, L)]
            p1 = pad_v[g, pl.ds(L, L)]
            nz = (p0 != 0).astype(jnp.int32) + (p1 != 0).astype(jnp.int32)
            cnt = jnp.maximum(jnp.sum(nz), 1)
            s = jnp.full((L,), 1.0, jnp.float32) / cnt.astype(jnp.float32)
            base = buf_off + e * N_TAGS
            row = e // 2
            col = (e % 2) * EMBED_DIM
            for k in range(EMBED_DIM // L):
                acc = tag_buf[base, pl.ds(k * L, L)]
                for t in range(1, N_TAGS):
                    acc = acc + tag_buf[base + t, pl.ds(k * L, L)]
                out_buf[row, pl.ds(col + k * L, L)] = acc * s
            return 0

        lax.fori_loop(0, CHUNK, elem_body, 0)
        pltpu.sync_copy(
            out_buf,
            out.at[pl.ds((wid * B_PER_W + c * CHUNK) // 2, CHUNK // 2)])

    issue(0, 0, sem_a)

    def step_body(st, _):
        c0 = 2 * st
        issue(c0 + 1, ROWS_PER_CHUNK, sem_b)
        drain(sem_a)
        compute(c0, 0)

        @pl.when(c0 + 2 < N_CHUNKS)
        def _():
            issue(c0 + 2, 0, sem_a)

        drain(sem_b)
        compute(c0 + 1, ROWS_PER_CHUNK)
        return 0

    lax.fori_loop(0, N_CHUNKS // 2, step_body, 0)


def _item_body(pairs, tagmean, jidx, halfoff, out, jv, hv, prow, tm_v, gsem):
    wid = lax.axis_index("s") * NC + lax.axis_index("c")

    pltpu.sync_copy(jidx.at[wid], jv)
    pltpu.sync_copy(halfoff.at[wid], hv)
    copies = []
    for q in range(B_PER_W // GATHER):
        copies.append(pltpu.async_copy(
            pairs.at[jv.at[q]], prow.at[pl.ds(q * GATHER, GATHER)], gsem))
    copies.append(pltpu.async_copy(
        tagmean.at[pl.ds(wid * (B_PER_W // 2), B_PER_W // 2)], tm_v, gsem))
    for cp in copies:
        cp.wait()

    def grp_body(grp, _):
        hv16 = hv[pl.ds(grp * L, L)]
        for i in range(L):
            e = grp * L + i
            co = hv16[i]
            for k in range(EMBED_DIM // L):
                v = (plsc.load_gather(
                        prow, [jnp.full((L,), e, jnp.int32),
                               co + k * L + lax.iota(jnp.int32, L)])
                     + tm_v[grp * (L // 2) + i // 2,
                            pl.ds((i % 2) * EMBED_DIM + k * L, L)])
                tm_v[grp * (L // 2) + i // 2,
                     pl.ds((i % 2) * EMBED_DIM + k * L, L)] = v
        return 0

    lax.fori_loop(0, B_PER_W // L, grp_body, 0)
    pltpu.sync_copy(tm_v, out.at[pl.ds(wid * (B_PER_W // 2), B_PER_W // 2)])


def _copy_body(x, o):
    o[...] = x[...]


@jax.jit
def _tc_only(item_t):
    return pl.pallas_call(
        _copy_body,
        grid=(TGRID,),
        in_specs=[pl.BlockSpec((EMBED_DIM, TW), lambda i: (0, i))],
        out_specs=pl.BlockSpec((EMBED_DIM, TW), lambda i: (0, i)),
        out_shape=jax.ShapeDtypeStruct((EMBED_DIM, TGRID * TW), jnp.float32),
    )(item_t)


@jax.jit
def _run(item_t, tags_emb, tags_idx, tags_pad, jidx, halfoff):
    tagmean = functools.partial(
        pl.kernel,
        out_type=jax.ShapeDtypeStruct((BATCH // 2, 2 * EMBED_DIM),
                                      jnp.float32),
        mesh=plsc.VectorSubcoreMesh(core_axis_name="c", subcore_axis_name="s"),
        scratch_types=[
            pltpu.VMEM((N_CHUNKS * G_PER_CHUNK, GATHER), jnp.int32),
            pltpu.VMEM((B_PER_W, 2 * L), jnp.int32),
            pltpu.VMEM((2 * ROWS_PER_CHUNK, EMBED_DIM), jnp.float32),
            pltpu.VMEM((CHUNK // 2, 2 * EMBED_DIM), jnp.float32),
            pltpu.SemaphoreType.DMA,
            pltpu.SemaphoreType.DMA,
        ],
        compiler_params=pltpu.CompilerParams(
            use_tc_tiling_on_sc=False, needs_layout_passes=False),
    )(_tags_body)(tags_emb, tags_idx, tags_pad)

    pairs = pl.pallas_call(
        _transpose_body,
        grid=(TGRID,),
        in_specs=[pl.BlockSpec((EMBED_DIM, TW), lambda i: (0, i))],
        out_specs=pl.BlockSpec((TW // 2, 2 * EMBED_DIM), lambda i: (i, 0)),
        out_shape=jax.ShapeDtypeStruct((PAIR_ROWS, 2 * EMBED_DIM),
                                       jnp.float32),
    )(item_t)

    return functools.partial(
        pl.kernel,
        out_type=jax.ShapeDtypeStruct((BATCH // 2, 2 * EMBED_DIM),
                                      jnp.float32),
        mesh=plsc.VectorSubcoreMesh(core_axis_name="c", subcore_axis_name="s"),
        scratch_types=[
            pltpu.VMEM((B_PER_W // GATHER, GATHER), jnp.int32),
            pltpu.VMEM((B_PER_W,), jnp.int32),
            pltpu.VMEM((B_PER_W, 2 * EMBED_DIM), jnp.float32),
            pltpu.VMEM((B_PER_W // 2, 2 * EMBED_DIM), jnp.float32),
            pltpu.SemaphoreType.DMA,
        ],
        compiler_params=pltpu.CompilerParams(
            use_tc_tiling_on_sc=True, needs_layout_passes=False),
    )(_item_body)(pairs, tagmean, jidx, halfoff)


def kernel(item_emb, item_tags_emb, item_ids, item_tags_ids):
    item_t = item_emb.T
    tags_idx = item_tags_ids.reshape(NW, N_CHUNKS * G_PER_CHUNK, GATHER)
    tags_pad = jnp.pad(item_tags_ids, ((0, 0), (0, 2 * L - N_TAGS))).reshape(
        NW, B_PER_W, 2 * L)
    jidx = ((item_ids // 256) * 128 + (item_ids % 128)).reshape(
        NW, B_PER_W // GATHER, GATHER)
    halfoff = (((item_ids % 256) // 128) * EMBED_DIM).astype(jnp.int32).reshape(
        NW, B_PER_W)
    out2 = _run(item_t, item_tags_emb, tags_idx, tags_pad, jidx, halfoff)
    return out2.reshape(BATCH, EMBED_DIM)


# speedup vs baseline: 1.0234x; 1.0234x over previous
import functools

import jax
import jax.numpy as jnp
from jax import lax
from jax.experimental import pallas as pl
from jax.experimental.pallas import tpu as pltpu
from jax.experimental.pallas import tpu_sc as plsc

EMBED_DIM = 64
ITEM_NUM = 1000000
HALF_ITEMS = ITEM_NUM // 2
BATCH = 16384
N_TAGS = 20

NC = 2
NS = 16
NW = NC * NS
B_PER_W = BATCH // NW
CHUNK = 32
N_CHUNKS = B_PER_W // CHUNK
ROWS_PER_CHUNK = CHUNK * N_TAGS
GATHER = 128
G_PER_CHUNK = ROWS_PER_CHUNK // GATHER
L = 16

TW = 32768
TGRID = -(-ITEM_NUM // TW)
PAIR_ROWS = TGRID * TW // 2


def _transpose_body(x, o):
    xv = x[...]
    parts = [jnp.swapaxes(xv[:, i * 128:(i + 1) * 128], 0, 1)
             for i in range(TW // 128)]
    o[...] = jnp.concatenate(
        [jnp.concatenate(parts[2 * r:2 * r + 2], axis=1)
         for r in range(TW // 256)], axis=0)


def _tags_body(tags_emb, tags_idx, tags_pad, out,
               idx_tags_v, pad_v, tag_buf, out_buf, gsem):
    wid = lax.axis_index("s") * NC + lax.axis_index("c")

    pltpu.sync_copy(tags_idx.at[wid], idx_tags_v)
    pltpu.sync_copy(tags_pad.at[wid], pad_v)

    def chunk_body(c, _):
        copies = []
        for j in range(G_PER_CHUNK):
            copies.append(pltpu.async_copy(
                tags_emb.at[idx_tags_v.at[c * G_PER_CHUNK + j]],
                tag_buf.at[pl.ds(j * GATHER, GATHER)], gsem))
        for cp in copies:
            cp.wait()

        def elem_body(e, _):
            g = c * CHUNK + e
            p0 = pad_v[g, pl.ds(0, L)]
            p1 = pad_v[g, pl.ds(L, L)]
            nz = (p0 != 0).astype(jnp.int32) + (p1 != 0).astype(jnp.int32)
            cnt = jnp.maximum(jnp.sum(nz), 1)
            s = jnp.full((L,), 1.0, jnp.float32) / cnt.astype(jnp.float32)
            base = e * N_TAGS
            row = e // 2
            col = (e % 2) * EMBED_DIM
            for k in range(EMBED_DIM // L):
                acc = tag_buf[base, pl.ds(k * L, L)]
                for t in range(1, N_TAGS):
                    acc = acc + tag_buf[base + t, pl.ds(k * L, L)]
                out_buf[row, pl.ds(col + k * L, L)] = acc * s
            return 0

        lax.fori_loop(0, CHUNK, elem_body, 0)
        pltpu.sync_copy(
            out_buf,
            out.at[pl.ds((wid * B_PER_W + c * CHUNK) // 2, CHUNK // 2)])
        return 0

    lax.fori_loop(0, N_CHUNKS, chunk_body, 0)


def _item_body(pairs, tagmean, jidx, halfoff, out, jv, hv, prow, tm_v, gsem):
    wid = lax.axis_index("s") * NC + lax.axis_index("c")

    pltpu.sync_copy(jidx.at[wid], jv)
    pltpu.sync_copy(halfoff.at[wid], hv)
    copies = []
    for q in range(B_PER_W // GATHER):
        copies.append(pltpu.async_copy(
            pairs.at[jv.at[q]], prow.at[pl.ds(q * GATHER, GATHER)], gsem))
    copies.append(pltpu.async_copy(
        tagmean.at[pl.ds(wid * (B_PER_W // 2), B_PER_W // 2)], tm_v, gsem))
    for cp in copies:
        cp.wait()

    def grp_body(grp, _):
        hv16 = hv[pl.ds(grp * L, L)]
        for i in range(L):
            e = grp * L + i
            co = hv16[i]
            for k in range(EMBED_DIM // L):
                v = (plsc.load_gather(
                        prow, [jnp.full((L,), e, jnp.int32),
                               co + k * L + lax.iota(jnp.int32, L)])
                     + tm_v[grp * (L // 2) + i // 2,
                            pl.ds((i % 2) * EMBED_DIM + k * L, L)])
                tm_v[grp * (L // 2) + i // 2,
                     pl.ds((i % 2) * EMBED_DIM + k * L, L)] = v
        return 0

    lax.fori_loop(0, B_PER_W // L, grp_body, 0)
    pltpu.sync_copy(tm_v, out.at[pl.ds(wid * (B_PER_W // 2), B_PER_W // 2)])


def _copy_body(x, o):
    o[...] = x[...]


@jax.jit
def _tc_only(item_t):
    return pl.pallas_call(
        _copy_body,
        grid=(TGRID,),
        in_specs=[pl.BlockSpec((EMBED_DIM, TW), lambda i: (0, i))],
        out_specs=pl.BlockSpec((EMBED_DIM, TW), lambda i: (0, i)),
        out_shape=jax.ShapeDtypeStruct((EMBED_DIM, TGRID * TW), jnp.float32),
    )(item_t)


@jax.jit
def _run(item_t, tags_emb, tags_idx, tags_pad, jidx, halfoff):
    tagmean = functools.partial(
        pl.kernel,
        out_type=jax.ShapeDtypeStruct((BATCH // 2, 2 * EMBED_DIM),
                                      jnp.float32),
        mesh=plsc.VectorSubcoreMesh(core_axis_name="c", subcore_axis_name="s"),
        scratch_types=[
            pltpu.VMEM((N_CHUNKS * G_PER_CHUNK, GATHER), jnp.int32),
            pltpu.VMEM((B_PER_W, 2 * L), jnp.int32),
            pltpu.VMEM((ROWS_PER_CHUNK, EMBED_DIM), jnp.float32),
            pltpu.VMEM((CHUNK // 2, 2 * EMBED_DIM), jnp.float32),
            pltpu.SemaphoreType.DMA,
        ],
        compiler_params=pltpu.CompilerParams(
            use_tc_tiling_on_sc=False, needs_layout_passes=False),
    )(_tags_body)(tags_emb, tags_idx, tags_pad)

    pairs = pl.pallas_call(
        _transpose_body,
        grid=(TGRID,),
        in_specs=[pl.BlockSpec((EMBED_DIM, TW), lambda i: (0, i))],
        out_specs=pl.BlockSpec((TW // 2, 2 * EMBED_DIM), lambda i: (i, 0)),
        out_shape=jax.ShapeDtypeStruct((PAIR_ROWS, 2 * EMBED_DIM),
                                       jnp.float32),
    )(item_t)

    return functools.partial(
        pl.kernel,
        out_type=jax.ShapeDtypeStruct((BATCH // 2, 2 * EMBED_DIM),
                                      jnp.float32),
        mesh=plsc.VectorSubcoreMesh(core_axis_name="c", subcore_axis_name="s"),
        scratch_types=[
            pltpu.VMEM((B_PER_W // GATHER, GATHER), jnp.int32),
            pltpu.VMEM((B_PER_W,), jnp.int32),
            pltpu.VMEM((B_PER_W, 2 * EMBED_DIM), jnp.float32),
            pltpu.VMEM((B_PER_W // 2, 2 * EMBED_DIM), jnp.float32),
            pltpu.SemaphoreType.DMA,
        ],
        compiler_params=pltpu.CompilerParams(
            use_tc_tiling_on_sc=True, needs_layout_passes=False),
    )(_item_body)(pairs, tagmean, jidx, halfoff)


def kernel(item_emb, item_tags_emb, item_ids, item_tags_ids):
    item_t = item_emb.T
    tags_idx = item_tags_ids.reshape(NW, N_CHUNKS * G_PER_CHUNK, GATHER)
    tags_pad = jnp.pad(item_tags_ids, ((0, 0), (0, 2 * L - N_TAGS))).reshape(
        NW, B_PER_W, 2 * L)
    jidx = ((item_ids // 256) * 128 + (item_ids % 128)).reshape(
        NW, B_PER_W // GATHER, GATHER)
    halfoff = (((item_ids % 256) // 128) * EMBED_DIM).astype(jnp.int32).reshape(
        NW, B_PER_W)
    out2 = _run(item_t, item_tags_emb, tags_idx, tags_pad, jidx, halfoff)
    return out2.reshape(BATCH, EMBED_DIM)
